# baseline (speedup 1.0000x reference)
import functools
import jax
import jax.numpy as jnp
from jax import lax
from jax.experimental import pallas as pl
from jax.experimental.pallas import tpu as pltpu
from jax.experimental.pallas import tpu_sc as plsc

B, S, D = 4, 4096, 2048
NC, NS = 2, 16
NW = NC * NS
S_PER_W = S // NW
CS = 8
CD = 256
NQ = D // CD
N_RCHUNK = S_PER_W // CS
NSLOT = 8
LOOK = 4


def _sc_add_body(x_hbm, emb_hbm, out_hbm, emb_v, x_v, *sems):
    in_sems = sems[:NSLOT]
    out_sems = sems[NSLOT:]
    wid = lax.axis_index("s") * NC + lax.axis_index("c")
    s_base = wid * S_PER_W

    def issue_in(row, col, slot):
        pltpu.async_copy(
            emb_hbm.at[pl.ds(row, CS), pl.ds(col, CD)], emb_v.at[slot],
            in_sems[slot],
        )
        for b in range(B):
            pltpu.async_copy(
                x_hbm.at[b, pl.ds(row, CS), pl.ds(col, CD)], x_v.at[slot, b],
                in_sems[slot],
            )

    def wait_in(row, col, slot):
        pltpu.make_async_copy(
            emb_hbm.at[pl.ds(row, CS), pl.ds(col, CD)], emb_v.at[slot],
            in_sems[slot],
        ).wait()
        for b in range(B):
            pltpu.make_async_copy(
                x_hbm.at[b, pl.ds(row, CS), pl.ds(col, CD)], x_v.at[slot, b],
                in_sems[slot],
            ).wait()

    def issue_out(row, col, slot):
        for b in range(B):
            pltpu.async_copy(
                x_v.at[slot, b], out_hbm.at[b, pl.ds(row, CS), pl.ds(col, CD)],
                out_sems[slot],
            )

    def wait_out(row, col, slot):
        for b in range(B):
            pltpu.make_async_copy(
                x_v.at[slot, b], out_hbm.at[b, pl.ds(row, CS), pl.ds(col, CD)],
                out_sems[slot],
            ).wait()

    def compute(slot):
        @plsc.parallel_loop(0, CD, step=16, unroll=2)
        def _(i):
            sl = pl.ds(i, 16)
            for r in range(CS):
                e = emb_v[slot, r, sl]
                for b in range(B):
                    plsc.addupdate(x_v.at[slot, b, r, sl], e)

    for q in range(LOOK):
        issue_in(s_base, q * CD, q)

    def t_body(t, _):
        row = s_base + t * CS
        for q in range(NQ):
            slot = q
            nslot = (q + LOOK) % NSLOT
            if q >= LOOK:
                wait_out(row, (q - LOOK) * CD, nslot)

                @pl.when(t < N_RCHUNK - 1)
                def _():
                    issue_in(row + CS, (q - LOOK) * CD, nslot)
            else:
                @pl.when(t >= 1)
                def _():
                    wait_out(row - CS, (q + LOOK) * CD, nslot)
                issue_in(row, (q + LOOK) * CD, nslot)
            wait_in(row, q * CD, slot)
            compute(slot)
            issue_out(row, q * CD, slot)
        return ()

    lax.fori_loop(0, N_RCHUNK, t_body, ())

    last_row = s_base + (N_RCHUNK - 1) * CS
    for q in range(NQ - LOOK, NQ):
        wait_out(last_row, q * CD, q % NSLOT)


@functools.partial(
    pl.kernel,
    out_type=jax.ShapeDtypeStruct((B, S, D), jnp.float32),
    mesh=plsc.VectorSubcoreMesh(core_axis_name="c", subcore_axis_name="s"),
    scratch_types=[
        pltpu.VMEM((NSLOT, CS, CD), jnp.float32),
        pltpu.VMEM((NSLOT, B, CS, CD), jnp.float32),
    ]
    + [pltpu.SemaphoreType.DMA] * (2 * NSLOT),
)
def _sc_add(x_hbm, emb_hbm, out_hbm, emb_v, x_v, *sems):
    _sc_add_body(x_hbm, emb_hbm, out_hbm, emb_v, x_v, *sems)


def kernel(x, emb_table):
    return _sc_add(x, emb_table)


# speedup vs baseline: 1.0874x; 1.0874x over previous
import functools
import jax
import jax.numpy as jnp
from jax import lax
from jax.experimental import pallas as pl
from jax.experimental.pallas import tpu as pltpu
from jax.experimental.pallas import tpu_sc as plsc

B, S, D = 4, 4096, 2048
NC, NS = 2, 16
NW = NC * NS
S_PER_W = S // NW
CS = 8
CD = 512
NQ = D // CD
N_RCHUNK = S_PER_W // CS
NSLOT = 4


def _sc_add_body(x_hbm, emb_hbm, out_hbm, emb_v, x_v, *sems):
    in_sems = sems[:NSLOT]
    out_sems = sems[NSLOT:]
    wid = lax.axis_index("s") * NC + lax.axis_index("c")
    s_base = wid * S_PER_W

    def issue_in(row, col, slot):
        pltpu.async_copy(
            emb_hbm.at[pl.ds(row, CS), pl.ds(col, CD)], emb_v.at[slot],
            in_sems[slot],
        )
        pltpu.async_copy(
            x_hbm.at[:, pl.ds(row, CS), pl.ds(col, CD)], x_v.at[slot],
            in_sems[slot],
        )

    def wait_in(row, col, slot):
        pltpu.make_async_copy(
            emb_hbm.at[pl.ds(row, CS), pl.ds(col, CD)], emb_v.at[slot],
            in_sems[slot],
        ).wait()
        pltpu.make_async_copy(
            x_hbm.at[:, pl.ds(row, CS), pl.ds(col, CD)], x_v.at[slot],
            in_sems[slot],
        ).wait()

    def issue_out(row, col, slot):
        pltpu.async_copy(
            x_v.at[slot], out_hbm.at[:, pl.ds(row, CS), pl.ds(col, CD)],
            out_sems[slot],
        )

    def wait_out(row, col, slot):
        pltpu.make_async_copy(
            x_v.at[slot], out_hbm.at[:, pl.ds(row, CS), pl.ds(col, CD)],
            out_sems[slot],
        ).wait()

    def compute(slot):
        @plsc.parallel_loop(0, CD, step=16, unroll=2)
        def _(i):
            sl = pl.ds(i, 16)
            for r in range(CS):
                e = emb_v[slot, r, sl]
                for b in range(B):
                    plsc.addupdate(x_v.at[slot, b, r, sl], e)

    def unit_pos(t, q):
        if q < NQ - 2:
            return t, (q + 2) * CD
        return t + 1, (q + 2 - NQ) * CD

    row0 = s_base
    issue_in(row0, 0, 0)
    issue_in(row0, CD, 1)

    def t_body(t, _):
        row = s_base + t * CS
        for q in range(NQ):
            slot = q
            nslot = (q + 2) % NSLOT
            nt, ncol = unit_pos(t, q)
            nrow = s_base + nt * CS
            if q >= 2:
                wait_out(row, (q - 2) * CD, nslot)

                @pl.when(t < N_RCHUNK - 1)
                def _():
                    issue_in(nrow, ncol, nslot)
            else:
                @pl.when(t >= 1)
                def _():
                    wait_out(s_base + (t - 1) * CS, (q + 2) * CD, nslot)
                issue_in(nrow, ncol, nslot)
            wait_in(row, q * CD, slot)
            compute(slot)
            issue_out(row, q * CD, slot)
        return ()

    lax.fori_loop(0, N_RCHUNK, t_body, ())

    last_row = s_base + (N_RCHUNK - 1) * CS
    wait_out(last_row, (NQ - 2) * CD, (NQ - 2) % NSLOT)
    wait_out(last_row, (NQ - 1) * CD, (NQ - 1) % NSLOT)


@functools.partial(
    pl.kernel,
    out_type=jax.ShapeDtypeStruct((B, S, D), jnp.float32),
    mesh=plsc.VectorSubcoreMesh(core_axis_name="c", subcore_axis_name="s"),
    scratch_types=[
        pltpu.VMEM((NSLOT, CS, CD), jnp.float32),
        pltpu.VMEM((NSLOT, B, CS, CD), jnp.float32),
    ]
    + [pltpu.SemaphoreType.DMA] * (2 * NSLOT),
)
def _sc_add(x_hbm, emb_hbm, out_hbm, emb_v, x_v, *sems):
    _sc_add_body(x_hbm, emb_hbm, out_hbm, emb_v, x_v, *sems)


def kernel(x, emb_table):
    return _sc_add(x, emb_table)
